# baseline (speedup 1.0000x reference)
import jax
import jax.numpy as jnp
from jax.experimental import pallas as pl
from jax.experimental.pallas import tpu as pltpu

_B, _N, _F, _FE, _FG, _H, _FOUT, _STEPS = 8, 128, 128, 16, 128, 128, 128, 4
_TI = 32
_NT = _N // _TI
_PB = 2


def _dot(a, b):
    return jax.lax.dot_general(a, b, (((1,), (0,)), ((), ())),
                               preferred_element_type=jnp.float32)


def _body(node_ref, edge_ref, graph_ref, adj_ref, wen_ref, w2_ref, weg_ref,
          m1_ref, m2_ref, mg_ref, o1_ref, o2_ref, wd_ref,
          out_ref, *me_scratches):
    bf16 = jnp.bfloat16
    w2 = w2_ref[...]
    wg = _dot(weg_ref[...], mg_ref[...])
    m1w, m2w, o1w, wd = m1_ref[...], m2_ref[...], o1_ref[...], wd_ref[...]

    for i in range(_PB):
        me_s = me_scratches[i]
        node_h = _dot(node_ref[i], wen_ref[...])
        mg = _dot(graph_ref[i], wg)

        edge = edge_ref[i]
        for t in range(_NT):
            blk = edge[t * _TI * 16:(t + 1) * _TI * 16].astype(bf16)
            me_s[t * _TI:(t + 1) * _TI] = _dot(blk, w2).astype(bf16).reshape(
                _TI, _N, _H)

        a1 = _dot(node_h, m1w[:_H])
        a2 = _dot(node_h, m2w[:_H]) + mg
        o1a = _dot(node_h, o1w[:_H])
        adj_f = adj_ref[i]
        adj = adj_f.astype(bf16)
        rs = jnp.sum(adj_f, axis=1, keepdims=True)

        hid = None
        for _ in range(_STEPS):
            if hid is None:
                m1, m2, hl = a1, a2, o1a
            else:
                m1 = a1 + _dot(hid, m1w[_H:])
                m2 = a2 + _dot(hid, m2w[_H:])
                hl = o1a + _dot(hid, o1w[_H:])
            nm1b, m2b = (-m1).astype(bf16), m2.astype(bf16)
            aggs = []
            for t in range(_NT):
                sl = slice(t * _TI, (t + 1) * _TI)
                msgs = jnp.maximum(
                    me_s[sl] + m2b[None, :, :],
                    nm1b[sl][:, None, :])
                aggs.append(jax.lax.dot_general(
                    adj[sl], msgs, (((1,), (1,)), ((0,), (0,))),
                    preferred_element_type=jnp.float32))
            agg = jnp.concatenate(aggs, axis=0) + m1 * rs
            hid = jnp.maximum(hl + _dot(agg, o2_ref[...]), 0.0)

        out_ref[i] = _dot(node_h, wd[:_H]) + _dot(hid, wd[_H:])


def kernel(node_fts, edge_fts, graph_fts, adj, W_enc_node, W_enc_edge,
           W_enc_graph, M1, M2, Me, Mg, O1, O2, W_dec):
    graph3 = graph_fts.reshape(_B, 1, _FG)
    edge2 = edge_fts.reshape(_B, _N * 16, 8 * _FE)
    w2 = jnp.kron(jnp.eye(8, dtype=jnp.float32),
                  jnp.dot(W_enc_edge, Me)).astype(jnp.bfloat16)
    wspec = lambda *shape: pl.BlockSpec(shape, lambda b: (0,) * len(shape))
    return pl.pallas_call(
        _body,
        grid=(_B // _PB,),
        in_specs=[
            pl.BlockSpec((_PB, _N, _F), lambda b: (b, 0, 0)),
            pl.BlockSpec((_PB, _N * 16, 8 * _FE), lambda b: (b, 0, 0)),
            pl.BlockSpec((_PB, 1, _FG), lambda b: (b, 0, 0)),
            pl.BlockSpec((_PB, _N, _N), lambda b: (b, 0, 0)),
            wspec(_F, _H),
            wspec(8 * _FE, 8 * _H),
            wspec(_FG, _H),
            wspec(2 * _H, _H),
            wspec(2 * _H, _H),
            wspec(_H, _H),
            wspec(2 * _H, _H),
            wspec(_H, _H),
            wspec(2 * _H, _FOUT),
        ],
        out_specs=pl.BlockSpec((_PB, _N, _FOUT), lambda b: (b, 0, 0)),
        out_shape=jax.ShapeDtypeStruct((_B, _N, _FOUT), jnp.float32),
        scratch_shapes=[pltpu.VMEM((_N, _N, _H), jnp.bfloat16)
                        for _ in range(_PB)],
        compiler_params=pltpu.CompilerParams(
            dimension_semantics=("arbitrary",)),
    )(node_fts, edge2, graph3, adj, W_enc_node, w2, W_enc_graph,
      M1, M2, Mg, O1, O2, W_dec)


# speedup vs baseline: 2.1702x; 2.1702x over previous
import jax
import jax.numpy as jnp
from jax.experimental import pallas as pl
from jax.experimental.pallas import tpu as pltpu

_B, _N, _F, _FE, _FG, _H, _FOUT, _STEPS = 8, 128, 128, 16, 128, 128, 128, 4
_TI = 32
_NT = _N // _TI
_PB = 2


def _dot(a, b):
    return jax.lax.dot_general(a, b, (((1,), (0,)), ((), ())),
                               preferred_element_type=jnp.float32)


def _body(node_ref, edge_ref, graph_ref, adj_ref, wen_ref, wee_ref, weg_ref,
          m1_ref, m2_ref, me_ref, mg_ref, o1_ref, o2_ref, wd_ref,
          out_ref, *me_scratches):
    bf16 = jnp.bfloat16
    wfe = _dot(wee_ref[...], me_ref[...]).astype(bf16)
    wg = _dot(weg_ref[...], mg_ref[...])
    wfe_b = jnp.broadcast_to(wfe[None], (_TI, _FE, _H))
    m1w, m2w, o1w, wd = m1_ref[...], m2_ref[...], o1_ref[...], wd_ref[...]

    for i in range(_PB):
        me_s = me_scratches[i]
        node_h = _dot(node_ref[i], wen_ref[...])
        mg = _dot(graph_ref[i], wg)

        edge = edge_ref[i]
        for t in range(_NT):
            blk = edge[t * _TI:(t + 1) * _TI]
            me_s[t * _TI:(t + 1) * _TI] = jax.lax.dot_general(
                blk, wfe_b, (((1,), (1,)), ((0,), (0,))),
                preferred_element_type=jnp.float32).astype(bf16)

        a1 = _dot(node_h, m1w[:_H])
        a2 = _dot(node_h, m2w[:_H]) + mg
        o1a = _dot(node_h, o1w[:_H])
        adj_f = adj_ref[i]
        adj = adj_f.astype(bf16)
        rs = jnp.sum(adj_f, axis=1, keepdims=True)

        hid = None
        for _ in range(_STEPS):
            if hid is None:
                m1, m2, hl = a1, a2, o1a
            else:
                m1 = a1 + _dot(hid, m1w[_H:])
                m2 = a2 + _dot(hid, m2w[_H:])
                hl = o1a + _dot(hid, o1w[_H:])
            nm1b, m2b = (-m1).astype(bf16), m2.astype(bf16)
            aggs = []
            for t in range(_NT):
                sl = slice(t * _TI, (t + 1) * _TI)
                msgs = jnp.maximum(
                    me_s[sl] + m2b[None, :, :],
                    nm1b[sl][:, None, :])
                aggs.append(jax.lax.dot_general(
                    adj[sl], msgs, (((1,), (1,)), ((0,), (0,))),
                    preferred_element_type=jnp.float32))
            agg = jnp.concatenate(aggs, axis=0) + m1 * rs
            hid = jnp.maximum(hl + _dot(agg, o2_ref[...]), 0.0)

        out_ref[i] = _dot(node_h, wd[:_H]) + _dot(hid, wd[_H:])


def kernel(node_fts, edge_fts, graph_fts, adj, W_enc_node, W_enc_edge,
           W_enc_graph, M1, M2, Me, Mg, O1, O2, W_dec):
    graph3 = graph_fts.reshape(_B, 1, _FG)
    edge_t = edge_fts.astype(jnp.bfloat16).transpose(0, 1, 3, 2)
    wspec = lambda *shape: pl.BlockSpec(shape, lambda b: (0,) * len(shape))
    return pl.pallas_call(
        _body,
        grid=(_B // _PB,),
        in_specs=[
            pl.BlockSpec((_PB, _N, _F), lambda b: (b, 0, 0)),
            pl.BlockSpec((_PB, _N, _FE, _N), lambda b: (b, 0, 0, 0)),
            pl.BlockSpec((_PB, 1, _FG), lambda b: (b, 0, 0)),
            pl.BlockSpec((_PB, _N, _N), lambda b: (b, 0, 0)),
            wspec(_F, _H),
            wspec(_FE, _H),
            wspec(_FG, _H),
            wspec(2 * _H, _H),
            wspec(2 * _H, _H),
            wspec(_H, _H),
            wspec(_H, _H),
            wspec(2 * _H, _H),
            wspec(_H, _H),
            wspec(2 * _H, _FOUT),
        ],
        out_specs=pl.BlockSpec((_PB, _N, _FOUT), lambda b: (b, 0, 0)),
        out_shape=jax.ShapeDtypeStruct((_B, _N, _FOUT), jnp.float32),
        scratch_shapes=[pltpu.VMEM((_N, _N, _H), jnp.bfloat16)
                        for _ in range(_PB)],
        compiler_params=pltpu.CompilerParams(
            dimension_semantics=("arbitrary",)),
    )(node_fts, edge_t, graph3, adj, W_enc_node, W_enc_edge, W_enc_graph,
      M1, M2, Me, Mg, O1, O2, W_dec)


# speedup vs baseline: 2.3875x; 1.1001x over previous
import jax
import jax.numpy as jnp
from jax.experimental import pallas as pl
from jax.experimental.pallas import tpu as pltpu

_B, _N, _F, _FE, _FG, _H, _FOUT, _STEPS = 8, 128, 128, 16, 128, 128, 128, 4
_TI = 32
_NT = _N // _TI
_PB = 2


def _dot(a, b):
    return jax.lax.dot_general(a, b, (((1,), (0,)), ((), ())),
                               preferred_element_type=jnp.float32)


def _body(node_ref, edge_ref, graph_ref, adj_ref, wen_ref, wee_ref, weg_ref,
          m1_ref, m2_ref, me_ref, mg_ref, o1_ref, o2_ref, wd_ref,
          out_ref, *me_scratches):
    bf16 = jnp.bfloat16
    wfe = _dot(wee_ref[...], me_ref[...]).astype(bf16)
    wg = _dot(weg_ref[...], mg_ref[...])
    wfe_b = jnp.broadcast_to(wfe[None], (_TI, _FE, _H))
    m1w, m2w, o1w, wd = m1_ref[...], m2_ref[...], o1_ref[...], wd_ref[...]

    R = range(_PB)
    node_h = [_dot(node_ref[i], wen_ref[...]) for i in R]
    mg = [_dot(graph_ref[i], wg) for i in R]

    for t in range(_NT):
        for i in R:
            blk = edge_ref[i][t * _TI:(t + 1) * _TI]
            me_scratches[i][t * _TI:(t + 1) * _TI] = jax.lax.dot_general(
                blk, wfe_b, (((1,), (1,)), ((0,), (0,))),
                preferred_element_type=jnp.float32).astype(bf16)

    a1 = [_dot(node_h[i], m1w[:_H]) for i in R]
    a2 = [_dot(node_h[i], m2w[:_H]) + mg[i] for i in R]
    o1a = [_dot(node_h[i], o1w[:_H]) for i in R]
    adj = [adj_ref[i].astype(bf16) for i in R]
    rs = [jnp.sum(adj_ref[i], axis=1, keepdims=True) for i in R]

    hid = [None] * _PB
    for s in range(_STEPS):
        m1, m2, hl = [None] * _PB, [None] * _PB, [None] * _PB
        for i in R:
            if s == 0:
                m1[i], m2[i], hl[i] = a1[i], a2[i], o1a[i]
            else:
                m1[i] = a1[i] + _dot(hid[i], m1w[_H:])
                m2[i] = a2[i] + _dot(hid[i], m2w[_H:])
                hl[i] = o1a[i] + _dot(hid[i], o1w[_H:])
        nm1b = [(-m1[i]).astype(bf16) for i in R]
        m2b = [m2[i].astype(bf16) for i in R]
        aggs = [[] for _ in R]
        for t in range(_NT):
            for i in R:
                sl = slice(t * _TI, (t + 1) * _TI)
                msgs = jnp.maximum(
                    me_scratches[i][sl] + m2b[i][None, :, :],
                    nm1b[i][sl][:, None, :])
                aggs[i].append(jax.lax.dot_general(
                    adj[i][sl], msgs, (((1,), (1,)), ((0,), (0,))),
                    preferred_element_type=jnp.float32))
        for i in R:
            agg = jnp.concatenate(aggs[i], axis=0) + m1[i] * rs[i]
            hid[i] = jnp.maximum(hl[i] + _dot(agg, o2_ref[...]), 0.0)

    for i in R:
        out_ref[i] = _dot(node_h[i], wd[:_H]) + _dot(hid[i], wd[_H:])


def kernel(node_fts, edge_fts, graph_fts, adj, W_enc_node, W_enc_edge,
           W_enc_graph, M1, M2, Me, Mg, O1, O2, W_dec):
    graph3 = graph_fts.reshape(_B, 1, _FG)
    edge_t = edge_fts.astype(jnp.bfloat16).transpose(0, 1, 3, 2)
    wspec = lambda *shape: pl.BlockSpec(shape, lambda b: (0,) * len(shape))
    return pl.pallas_call(
        _body,
        grid=(_B // _PB,),
        in_specs=[
            pl.BlockSpec((_PB, _N, _F), lambda b: (b, 0, 0)),
            pl.BlockSpec((_PB, _N, _FE, _N), lambda b: (b, 0, 0, 0)),
            pl.BlockSpec((_PB, 1, _FG), lambda b: (b, 0, 0)),
            pl.BlockSpec((_PB, _N, _N), lambda b: (b, 0, 0)),
            wspec(_F, _H),
            wspec(_FE, _H),
            wspec(_FG, _H),
            wspec(2 * _H, _H),
            wspec(2 * _H, _H),
            wspec(_H, _H),
            wspec(_H, _H),
            wspec(2 * _H, _H),
            wspec(_H, _H),
            wspec(2 * _H, _FOUT),
        ],
        out_specs=pl.BlockSpec((_PB, _N, _FOUT), lambda b: (b, 0, 0)),
        out_shape=jax.ShapeDtypeStruct((_B, _N, _FOUT), jnp.float32),
        scratch_shapes=[pltpu.VMEM((_N, _N, _H), jnp.bfloat16)
                        for _ in range(_PB)],
        compiler_params=pltpu.CompilerParams(
            dimension_semantics=("arbitrary",)),
    )(node_fts, edge_t, graph3, adj, W_enc_node, W_enc_edge, W_enc_graph,
      M1, M2, Me, Mg, O1, O2, W_dec)


# speedup vs baseline: 2.4766x; 1.0373x over previous
import jax
import jax.numpy as jnp
from jax.experimental import pallas as pl
from jax.experimental.pallas import tpu as pltpu

_B, _N, _F, _FE, _FG, _H, _FOUT, _STEPS = 8, 128, 128, 16, 128, 128, 128, 4
_TI = 32
_NT = _N // _TI
_PB = 4


def _dot(a, b):
    return jax.lax.dot_general(a, b, (((1,), (0,)), ((), ())),
                               preferred_element_type=jnp.float32)


def _body(node_ref, edge_ref, graph_ref, adj_ref, wen_ref, wee_ref, weg_ref,
          m1_ref, m2_ref, me_ref, mg_ref, o1_ref, o2_ref, wd_ref,
          out_ref, *me_scratches):
    bf16 = jnp.bfloat16
    wfe = _dot(wee_ref[...], me_ref[...]).astype(bf16)
    wg = _dot(weg_ref[...], mg_ref[...])
    wfe_b = jnp.broadcast_to(wfe[None], (_TI, _FE, _H))
    m1w, m2w, o1w, wd = m1_ref[...], m2_ref[...], o1_ref[...], wd_ref[...]

    R = range(_PB)
    node_h = [_dot(node_ref[i], wen_ref[...]) for i in R]
    mg = [_dot(graph_ref[i], wg) for i in R]

    for t in range(_NT):
        for i in R:
            blk = edge_ref[i][t * _TI:(t + 1) * _TI]
            me_scratches[i][t * _TI:(t + 1) * _TI] = jax.lax.dot_general(
                blk, wfe_b, (((1,), (1,)), ((0,), (0,))),
                preferred_element_type=jnp.float32).astype(bf16)

    a1 = [_dot(node_h[i], m1w[:_H]) for i in R]
    a2 = [_dot(node_h[i], m2w[:_H]) + mg[i] for i in R]
    o1a = [_dot(node_h[i], o1w[:_H]) for i in R]
    adj = [adj_ref[i].astype(bf16) for i in R]
    rs = [jnp.sum(adj_ref[i], axis=1, keepdims=True) for i in R]

    hid = [None] * _PB
    for s in range(_STEPS):
        m1, m2, hl = [None] * _PB, [None] * _PB, [None] * _PB
        for i in R:
            if s == 0:
                m1[i], m2[i], hl[i] = a1[i], a2[i], o1a[i]
            else:
                m1[i] = a1[i] + _dot(hid[i], m1w[_H:])
                m2[i] = a2[i] + _dot(hid[i], m2w[_H:])
                hl[i] = o1a[i] + _dot(hid[i], o1w[_H:])
        nm1b = [(-m1[i]).astype(bf16) for i in R]
        m2b = [m2[i].astype(bf16) for i in R]
        aggs = [[] for _ in R]
        for t in range(_NT):
            for i in R:
                sl = slice(t * _TI, (t + 1) * _TI)
                msgs = jnp.maximum(
                    me_scratches[i][sl] + m2b[i][None, :, :],
                    nm1b[i][sl][:, None, :])
                aggs[i].append(jax.lax.dot_general(
                    adj[i][sl], msgs, (((1,), (1,)), ((0,), (0,))),
                    preferred_element_type=jnp.float32))
        for i in R:
            agg = jnp.concatenate(aggs[i], axis=0) + m1[i] * rs[i]
            hid[i] = jnp.maximum(hl[i] + _dot(agg, o2_ref[...]), 0.0)

    for i in R:
        out_ref[i] = _dot(node_h[i], wd[:_H]) + _dot(hid[i], wd[_H:])


def kernel(node_fts, edge_fts, graph_fts, adj, W_enc_node, W_enc_edge,
           W_enc_graph, M1, M2, Me, Mg, O1, O2, W_dec):
    graph3 = graph_fts.reshape(_B, 1, _FG)
    edge_t = edge_fts.astype(jnp.bfloat16).transpose(0, 1, 3, 2)
    wspec = lambda *shape: pl.BlockSpec(shape, lambda b: (0,) * len(shape))
    return pl.pallas_call(
        _body,
        grid=(_B // _PB,),
        in_specs=[
            pl.BlockSpec((_PB, _N, _F), lambda b: (b, 0, 0)),
            pl.BlockSpec((_PB, _N, _FE, _N), lambda b: (b, 0, 0, 0)),
            pl.BlockSpec((_PB, 1, _FG), lambda b: (b, 0, 0)),
            pl.BlockSpec((_PB, _N, _N), lambda b: (b, 0, 0)),
            wspec(_F, _H),
            wspec(_FE, _H),
            wspec(_FG, _H),
            wspec(2 * _H, _H),
            wspec(2 * _H, _H),
            wspec(_H, _H),
            wspec(_H, _H),
            wspec(2 * _H, _H),
            wspec(_H, _H),
            wspec(2 * _H, _FOUT),
        ],
        out_specs=pl.BlockSpec((_PB, _N, _FOUT), lambda b: (b, 0, 0)),
        out_shape=jax.ShapeDtypeStruct((_B, _N, _FOUT), jnp.float32),
        scratch_shapes=[pltpu.VMEM((_N, _N, _H), jnp.bfloat16)
                        for _ in range(_PB)],
        compiler_params=pltpu.CompilerParams(
            dimension_semantics=("arbitrary",)),
    )(node_fts, edge_t, graph3, adj, W_enc_node, W_enc_edge, W_enc_graph,
      M1, M2, Me, Mg, O1, O2, W_dec)
